# baseline (speedup 1.0000x reference)
import functools

import jax
import jax.numpy as jnp
from jax import lax
from jax.experimental import pallas as pl
from jax.experimental.pallas import tpu as pltpu
from jax.experimental.pallas import tpu_sc as plsc

VOCAB = 1000
EMBED_DIM = 128
REL = 5
LANES = 16
NC = 2
NS = 16
NW = NC * NS


def _project_body(w_ref, table_ref, b_ref, out_ref):
    w = w_ref[...]
    t = table_ref[...]
    p = lax.dot_general(w, t, (((1,), (1,)), ((), ())),
                        preferred_element_type=jnp.float32)
    out_ref[...] = jnp.maximum(p + b_ref[...], 0.0)


def _project_t(table, W, b):
    return pl.pallas_call(
        _project_body,
        out_shape=jax.ShapeDtypeStruct((REL, VOCAB), jnp.float32),
    )(W, table, b.reshape(REL, 1))


NBUF = 2


def _make_gather(L: int, B: int):
    bw = B // NW
    n_lt = L // 8
    n_full = (n_lt // NBUF) * NBUF
    mesh = plsc.VectorSubcoreMesh(core_axis_name="c", subcore_axis_name="s")

    @functools.partial(
        pl.kernel, mesh=mesh,
        out_type=jax.ShapeDtypeStruct((REL, L, B), jnp.float32),
        compiler_params=pltpu.CompilerParams(needs_layout_passes=False),
        scratch_types=(
            [pltpu.VMEM((REL * VOCAB,), jnp.float32),
             pltpu.VMEM((NBUF, 8, bw), jnp.int32),
             pltpu.VMEM((NBUF, REL, 8, bw), jnp.float32)]
            + [pltpu.SemaphoreType.DMA] * (2 * NBUF)
        ),
    )
    def gather(pt_hbm, ctx_hbm, out_hbm, pt_v, idx_v, out_v, *allsems):
        wid = lax.axis_index("s") * NC + lax.axis_index("c")
        b0 = wid * bw
        sems = allsems[:NBUF]
        isems = allsems[NBUF:]

        def in_slices(lt, buf):
            return ctx_hbm.at[pl.ds(lt * 8, 8), pl.ds(b0, bw)], idx_v.at[buf]

        def out_slices(lt, buf):
            return out_v.at[buf], out_hbm.at[:, pl.ds(lt * 8, 8),
                                             pl.ds(b0, bw)]

        for b in range(NBUF):
            pltpu.async_copy(*in_slices(b, b), isems[b])
        pltpu.sync_copy(pt_hbm, pt_v)

        def chunk_body(lt, buf, first, pre_lt=None, pre_guard=None):
            src_i, dst_i = in_slices(lt, buf)
            pltpu.make_async_copy(src_i, dst_i, isems[buf]).wait()
            src, dst = out_slices(lt, buf)
            if not first:
                pltpu.make_async_copy(src, dst, sems[buf]).wait()
            GB = 8
            for s in range(8):
                def do_group(g, carry):
                    k0 = g * (GB * LANES)
                    idxs = [idx_v[buf, s, pl.ds(k0 + j * LANES, LANES)]
                            for j in range(GB)]
                    vals = [[plsc.load_gather(
                        pt_v.at[pl.ds(c * VOCAB, VOCAB)], [idxs[j]])
                        for c in range(REL)] for j in range(GB)]
                    for j in range(GB):
                        for c in range(REL):
                            out_v[buf, c, s,
                                  pl.ds(k0 + j * LANES, LANES)] = vals[j][c]
                    return carry

                lax.fori_loop(0, bw // (GB * LANES), do_group, 0)
            pltpu.async_copy(src, dst, sems[buf])
            if pre_lt is not None:
                def issue():
                    pltpu.async_copy(*in_slices(pre_lt, buf), isems[buf])
                if pre_guard is not None:
                    pl.when(pre_guard)(issue)
                else:
                    issue()

        for b in range(NBUF):
            chunk_body(b, b, True, pre_lt=b + NBUF)

        def do_round(i, carry):
            for b in range(NBUF):
                lt = NBUF * i + b
                pre = lt + NBUF
                chunk_body(lt, b, False, pre_lt=pre,
                           pre_guard=(pre < n_lt) if b else None)
            return carry

        lax.fori_loop(1, n_full // NBUF, do_round, 0)
        for lt in range(n_full, n_lt):
            chunk_body(lt, lt - n_full, False)
        for b in range(NBUF):
            src, dst = out_slices(0, b)
            pltpu.make_async_copy(src, dst, sems[b]).wait()

    return gather


def kernel(context, table, W, b):
    B, L = context.shape
    Pt = _project_t(table, W, b)
    ctx_t = context.T
    out5 = _make_gather(L, B)(Pt.reshape(REL * VOCAB), ctx_t)
    return jnp.transpose(out5, (2, 1, 0))


# speedup vs baseline: 1.1722x; 1.1722x over previous
import functools

import jax
import jax.numpy as jnp
from jax import lax
from jax.experimental import pallas as pl
from jax.experimental.pallas import tpu as pltpu
from jax.experimental.pallas import tpu_sc as plsc

VOCAB = 1000
EMBED_DIM = 128
REL = 5
LANES = 16
NC = 2
NS = 16
NW = NC * NS


def _project_body(w_ref, table_ref, b_ref, out_ref):
    w = w_ref[...]
    t = table_ref[...]
    p = lax.dot_general(w, t, (((1,), (1,)), ((), ())),
                        preferred_element_type=jnp.float32)
    out_ref[...] = jnp.maximum(p + b_ref[...], 0.0)


def _project_t(table, W, b):
    return pl.pallas_call(
        _project_body,
        out_shape=jax.ShapeDtypeStruct((REL, VOCAB), jnp.float32),
    )(W, table, b.reshape(REL, 1))


NBUF = 2


def _make_gather(L: int, B: int):
    bw = B // NW
    n_lt = L // 8
    n_full = (n_lt // NBUF) * NBUF
    mesh = plsc.VectorSubcoreMesh(core_axis_name="c", subcore_axis_name="s")

    @functools.partial(
        pl.kernel, mesh=mesh,
        out_type=jax.ShapeDtypeStruct((REL, L, B), jnp.float32),
        compiler_params=pltpu.CompilerParams(needs_layout_passes=False),
        scratch_types=(
            [pltpu.VMEM((REL * VOCAB,), jnp.float32),
             pltpu.VMEM((NBUF, 8, bw), jnp.int32),
             pltpu.VMEM((NBUF, REL, 8, bw), jnp.float32)]
            + [pltpu.SemaphoreType.DMA] * (2 * NBUF)
        ),
    )
    def gather(pt_hbm, ctx_hbm, out_hbm, pt_v, idx_v, out_v, *allsems):
        wid = lax.axis_index("s") * NC + lax.axis_index("c")
        b0 = wid * bw
        sems = allsems[:NBUF]
        isems = allsems[NBUF:]

        def in_slices(lt, buf):
            return ctx_hbm.at[pl.ds(lt * 8, 8), pl.ds(b0, bw)], idx_v.at[buf]

        def out_slices(lt, buf):
            return out_v.at[buf], out_hbm.at[:, pl.ds(lt * 8, 8),
                                             pl.ds(b0, bw)]

        for b in range(NBUF):
            pltpu.async_copy(*in_slices(b, b), isems[b])
        pltpu.sync_copy(pt_hbm, pt_v)

        def chunk_body(lt, buf, first, pre_lt=None, pre_guard=None):
            src_i, dst_i = in_slices(lt, buf)
            pltpu.make_async_copy(src_i, dst_i, isems[buf]).wait()
            src, dst = out_slices(lt, buf)
            if not first:
                pltpu.make_async_copy(src, dst, sems[buf]).wait()
            GB = 4
            for s in range(8):
                def do_group(g, carry):
                    k0 = g * (GB * LANES)
                    idxs = [idx_v[buf, s, pl.ds(k0 + j * LANES, LANES)]
                            for j in range(GB)]
                    vals = [[plsc.load_gather(
                        pt_v.at[pl.ds(c * VOCAB, VOCAB)], [idxs[j]])
                        for c in range(REL)] for j in range(GB)]
                    for j in range(GB):
                        for c in range(REL):
                            out_v[buf, c, s,
                                  pl.ds(k0 + j * LANES, LANES)] = vals[j][c]
                    return carry

                lax.fori_loop(0, bw // (GB * LANES), do_group, 0)
            pltpu.async_copy(src, dst, sems[buf])
            if pre_lt is not None:
                def issue():
                    pltpu.async_copy(*in_slices(pre_lt, buf), isems[buf])
                if pre_guard is not None:
                    pl.when(pre_guard)(issue)
                else:
                    issue()

        for b in range(NBUF):
            chunk_body(b, b, True, pre_lt=b + NBUF)

        def do_round(i, carry):
            for b in range(NBUF):
                lt = NBUF * i + b
                pre = lt + NBUF
                chunk_body(lt, b, False, pre_lt=pre,
                           pre_guard=(pre < n_lt) if b else None)
            return carry

        lax.fori_loop(1, n_full // NBUF, do_round, 0)
        for lt in range(n_full, n_lt):
            chunk_body(lt, lt - n_full, False)
        for b in range(NBUF):
            src, dst = out_slices(0, b)
            pltpu.make_async_copy(src, dst, sems[b]).wait()

    return gather


def kernel(context, table, W, b):
    B, L = context.shape
    Pt = _project_t(table, W, b)
    ctx_t = context.T
    out5 = _make_gather(L, B)(Pt.reshape(REL * VOCAB), ctx_t)
    return jnp.transpose(out5, (2, 1, 0))
